# baseline (speedup 1.0000x reference)
import itertools as it

import jax
import jax.numpy as jnp
from jax import lax
from jax.experimental import pallas as pl
from jax.experimental.pallas import tpu as pltpu
from jax.experimental.pallas import tpu_sc as plsc

DIM = 4
ROWS = 819200
ROW = 17
N = ROWS * ROW
NW = 32
ROWS_PER_W = ROWS // NW
PIECE_ROWS = 1600
PIECE = PIECE_ROWS * ROW
PIECES = ROWS_PER_W // PIECE_ROWS
PERIOD = 272
UNROLL = 2


def _perm_idx():
    permutations = jnp.array(list(it.permutations(range(DIM))), dtype=jnp.int32)
    num_perms, num_ue = permutations.shape
    key = jax.random.key(42)
    _p = jax.random.randint(key, (1,), 0, num_perms, dtype=jnp.int32)
    perm = permutations[_p[0], :]
    t = jnp.tile(perm, num_ue)
    r = jnp.repeat(perm, num_ue, axis=0)
    idx = num_ue * r + t
    return jnp.concatenate((idx, jnp.array([num_ue ** 2], dtype=jnp.int32)))


def _permute_sc(x_flat, src0):
    mesh = plsc.VectorSubcoreMesh(core_axis_name="c", subcore_axis_name="s")

    @pl.kernel(
        out_type=jax.ShapeDtypeStruct((N,), jnp.float32),
        mesh=mesh,
        compiler_params=pltpu.CompilerParams(
            needs_layout_passes=False, use_tc_tiling_on_sc=True),
        scratch_types=[
            pltpu.VMEM((PIECE,), jnp.float32),
            pltpu.VMEM((PIECE,), jnp.float32),
            pltpu.VMEM((PIECE,), jnp.float32),
            pltpu.VMEM((PIECE,), jnp.float32),
            pltpu.VMEM((PERIOD,), jnp.int32),
            pltpu.SemaphoreType.DMA,
            pltpu.SemaphoreType.DMA,
            pltpu.SemaphoreType.DMA,
            pltpu.SemaphoreType.DMA,
        ],
    )
    def body(x_hbm, src_hbm, out_hbm, in0, in1, o0, o1, idx_v,
             isem0, isem1, osem0, osem1):
        wid = lax.axis_index("s") * 2 + lax.axis_index("c")
        woff = wid * (ROWS_PER_W * ROW)
        pltpu.sync_copy(src_hbm, idx_v)
        pats = [idx_v[pl.ds(j * 16, 16)] for j in range(ROW)]
        ins, outs = (in0, in1), (o0, o1)
        isems, osems = (isem0, isem1), (osem0, osem1)

        def in_copy(p):
            b = p % 2
            return pltpu.make_async_copy(
                x_hbm.at[pl.ds(woff + p * PIECE, PIECE)], ins[b], isems[b])

        def out_copy(p):
            b = p % 2
            return pltpu.make_async_copy(
                outs[b], out_hbm.at[pl.ds(woff + p * PIECE, PIECE)], osems[b])

        in_copy(0).start()
        for p in range(PIECES):
            b = p % 2
            in_copy(p).wait()
            if p + 1 < PIECES:
                in_copy(p + 1).start()
            if p >= 2:
                out_copy(p - 2).wait()
            in_b, out_b = ins[b], outs[b]

            @plsc.parallel_loop(0, PIECE, PERIOD, unroll=UNROLL)
            def blk(base):
                view = in_b.at[pl.ds(base, PERIOD)]
                for j in range(ROW):
                    out_b[pl.ds(base + j * 16, 16)] = plsc.load_gather(
                        view, [pats[j]])

            out_copy(p).start()
        out_copy(PIECES - 2).wait()
        out_copy(PIECES - 1).wait()

    return body(x_flat, src0)


NQ = 2


def _permute_tc(xv, idx_eff):
    SP = 25
    H = 200 // SP
    PIECES = ROW * SP
    NB = 32
    LAG = 16

    def body(idx_ref, x_ref, o_ref, *scratch):
        bufs = scratch[:NB]
        isems = scratch[NB:2 * NB]
        osems = scratch[2 * NB:3 * NB]

        def incpy(i):
            j, h = i // SP, i % SP
            return pltpu.make_async_copy(
                x_ref.at[idx_ref[j], pl.ds(h * H, H)],
                bufs[i % NB], isems[i % NB])

        def outcpy(i):
            j, h = i // SP, i % SP
            return pltpu.make_async_copy(
                bufs[i % NB], o_ref.at[j, pl.ds(h * H, H)], osems[i % NB])

        out_waited = [False] * PIECES
        for i in range(NB):
            incpy(i).start(priority=i % NQ)
        for i in range(PIECES):
            incpy(i).wait()
            outcpy(i).start(priority=i % NQ)
            r = i + NB - LAG
            if NB <= r < PIECES:
                outcpy(r - NB).wait()
                out_waited[r - NB] = True
                incpy(r).start(priority=r % NQ)
        for i in range(PIECES):
            if not out_waited[i]:
                outcpy(i).wait()

    return pl.pallas_call(
        body,
        in_specs=[
            pl.BlockSpec(memory_space=pltpu.SMEM),
            pl.BlockSpec(memory_space=pl.ANY),
        ],
        out_specs=pl.BlockSpec(memory_space=pl.ANY),
        out_shape=jax.ShapeDtypeStruct((ROW, 200, 4096), jnp.float32),
        scratch_shapes=(
            [pltpu.VMEM((H, 4096), jnp.float32) for _ in range(NB)]
            + [pltpu.SemaphoreType.DMA for _ in range(2 * NB)]
        ),
    )(idx_eff, xv)


def kernel(x, training):
    perm_idx = _perm_idx()
    idx_eff = jnp.where(training != 0, perm_idx,
                        jnp.arange(ROW, dtype=jnp.int32))
    xv = jnp.transpose(x, (2, 1, 0))
    ov = _permute_tc(xv, idx_eff)
    return jnp.transpose(ov, (2, 1, 0))


# speedup vs baseline: 1.1520x; 1.1520x over previous
import itertools as it

import jax
import jax.numpy as jnp
from jax.experimental import pallas as pl
from jax.experimental.pallas import tpu as pltpu

DIM = 4
ROW = 17
NQ = 2


def _perm_idx():
    permutations = jnp.array(list(it.permutations(range(DIM))), dtype=jnp.int32)
    num_perms, num_ue = permutations.shape
    key = jax.random.key(42)
    _p = jax.random.randint(key, (1,), 0, num_perms, dtype=jnp.int32)
    perm = permutations[_p[0], :]
    t = jnp.tile(perm, num_ue)
    r = jnp.repeat(perm, num_ue, axis=0)
    idx = num_ue * r + t
    return jnp.concatenate((idx, jnp.array([num_ue ** 2], dtype=jnp.int32)))


def _permute_tc(xv, idx_eff):
    SP = 5
    H = 200 // SP
    PIECES = ROW * SP
    NB = 24
    LAG = 12

    def body(idx_ref, x_ref, o_ref, *scratch):
        bufs = scratch[:NB]
        isems = scratch[NB:2 * NB]
        osems = scratch[2 * NB:3 * NB]

        def incpy(i):
            j, h = i // SP, i % SP
            return pltpu.make_async_copy(
                x_ref.at[idx_ref[j], pl.ds(h * H, H)],
                bufs[i % NB], isems[i % NB])

        def outcpy(i):
            j, h = i // SP, i % SP
            return pltpu.make_async_copy(
                bufs[i % NB], o_ref.at[j, pl.ds(h * H, H)], osems[i % NB])

        out_waited = [False] * PIECES
        for i in range(NB):
            incpy(i).start(priority=i % NQ)
        for i in range(PIECES):
            incpy(i).wait()
            outcpy(i).start(priority=i % NQ)
            r = i + NB - LAG
            if NB <= r < PIECES:
                outcpy(r - NB).wait()
                out_waited[r - NB] = True
                incpy(r).start(priority=r % NQ)
        for i in range(PIECES):
            if not out_waited[i]:
                outcpy(i).wait()

    return pl.pallas_call(
        body,
        in_specs=[
            pl.BlockSpec(memory_space=pltpu.SMEM),
            pl.BlockSpec(memory_space=pl.ANY),
        ],
        out_specs=pl.BlockSpec(memory_space=pl.ANY),
        out_shape=jax.ShapeDtypeStruct((ROW, 200, 4096), jnp.float32),
        scratch_shapes=(
            [pltpu.VMEM((H, 4096), jnp.float32) for _ in range(NB)]
            + [pltpu.SemaphoreType.DMA for _ in range(2 * NB)]
        ),
    )(idx_eff, xv)


def kernel(x, training):
    perm_idx = _perm_idx()
    idx_eff = jnp.where(training != 0, perm_idx,
                        jnp.arange(ROW, dtype=jnp.int32))
    xv = jnp.transpose(x, (2, 1, 0))
    ov = _permute_tc(xv, idx_eff)
    return jnp.transpose(ov, (2, 1, 0))
